# baseline (speedup 1.0000x reference)
import functools

import numpy as np
import jax
from jax import lax
import jax.numpy as jnp
from jax.experimental import pallas as pl
from jax.experimental.pallas import tpu as pltpu
from jax.experimental.pallas import tpu_sc as plsc

N = 10000
E = 320000
D = 128
H = 16
EF = 4
G = 20
RF = 80
DH = D // H
TE = 512
XP = 128
TDW = 2 * D + XP
TSW = D + XP

NW = 32
EPW = E // NW
GC = 80
NCH = EPW // GC

NACC1 = 11264
NACC2 = 10240

@functools.cache
def _sc_mesh():
    return plsc.VectorSubcoreMesh(core_axis_name="c", subcore_axis_name="s")


def _gather_body(td_hbm, ts_hbm, dsti_hbm, srci_hbm, gd_hbm, gs_hbm,
                 idx_d, idx_s, rows_d, rows_s, sem):
    wid = lax.axis_index("s") * 2 + lax.axis_index("c")
    pltpu.sync_copy(dsti_hbm.at[wid], idx_d)
    pltpu.sync_copy(srci_hbm.at[wid], idx_s)

    def chunk(j, carry):
        base = wid * EPW + j * GC
        cp1 = pltpu.async_copy(td_hbm.at[idx_d.at[j]], rows_d, sem)
        cp2 = pltpu.async_copy(ts_hbm.at[idx_s.at[j]], rows_s, sem)
        cp1.wait()
        cp2.wait()
        pltpu.sync_copy(rows_d, gd_hbm.at[pl.ds(base, GC)])
        pltpu.sync_copy(rows_s, gs_hbm.at[pl.ds(base, GC)])
        return carry

    lax.fori_loop(0, NCH, chunk, 0)


@functools.cache
def _build_sc_gather():
    return functools.partial(
        pl.kernel, _gather_body,
        mesh=_sc_mesh(),
        out_type=(jax.ShapeDtypeStruct((E, TDW), jnp.float32),
                  jax.ShapeDtypeStruct((E, TSW), jnp.float32)),
        scratch_types=[
            pltpu.VMEM((NCH, GC), jnp.int32),
            pltpu.VMEM((NCH, GC), jnp.int32),
            pltpu.VMEM((GC, TDW), jnp.float32),
            pltpu.VMEM((GC, TSW), jnp.float32),
            pltpu.SemaphoreType.DMA,
        ],
    )()


def _sc_gather(*args):
    return _build_sc_gather()(*args)


def _make_scatter(nacc, nstreams):
    rpw = nacc // 16

    def body(*refs):
        ins = refs[:2 * nstreams]
        zeros_hbm = refs[2 * nstreams]
        out_hbm = refs[2 * nstreams + 1]
        scr = refs[2 * nstreams + 2:]
        idx = scr[0]
        rows = scr[1]
        acc = scr[2]

        cid = lax.axis_index("c")
        sid = lax.axis_index("s")
        wid = sid * 2 + cid
        pltpu.sync_copy(zeros_hbm, acc.at[pl.ds(sid * rpw, rpw)])
        plsc.subcore_barrier()

        def chunk(j, carry):
            base = wid * EPW + j * GC
            for t in range(nstreams):
                pltpu.sync_copy(ins[2 * t + 1].at[wid, j], idx)
                pltpu.sync_copy(ins[2 * t].at[pl.ds(base, GC)], rows)
                pltpu.sync_copy(rows, acc.at[idx], add=True)
            return carry

        lax.fori_loop(0, NCH, chunk, 0)
        plsc.subcore_barrier()
        pltpu.sync_copy(acc.at[pl.ds(sid * rpw, rpw)],
                        out_hbm.at[cid, pl.ds(sid * rpw, rpw)])

    scratch = [pltpu.VMEM((GC,), jnp.int32),
               pltpu.VMEM((GC, 128), jnp.float32),
               pltpu.VMEM_SHARED((nacc, 128), jnp.float32)]
    return functools.partial(
        pl.kernel, body,
        mesh=_sc_mesh(),
        out_type=jax.ShapeDtypeStruct((2, nacc, 128), jnp.float32),
        scratch_types=scratch,
    )()


_make_scatter = functools.cache(_make_scatter)


def _scatter1(*args):
    return _make_scatter(NACC1, 2)(*args)


def _scatter2(*args):
    return _make_scatter(NACC2, 1)(*args)


def _ln_relu(z, g, bt):
    mu = jnp.mean(z, axis=-1, keepdims=True)
    zc = z - mu
    var = jnp.mean(zc * zc, axis=-1, keepdims=True)
    zn = zc * jax.lax.rsqrt(var + 1e-5) * g + bt
    return jnp.maximum(zn, 0.0)


def _edge_common(ea, gd, gs, wk, wv, eww, ewb):
    f32 = jnp.float32
    hd = gd[:, 0:D]
    qd = gd[:, D:2 * D]
    xd = gd[:, 2 * D:2 * D + 3]
    hs = gs[:, 0:D]
    xs = gs[:, D:D + 3]

    rel = xd - xs
    dist = jnp.sqrt(jnp.sum(rel * rel, axis=-1, keepdims=True) + 1e-12)
    step = 10.0 / (G - 1)
    coeff = -0.5 / step ** 2
    offs = jax.lax.broadcasted_iota(jnp.int32, (1, G), 1).astype(f32) * step
    df = jnp.exp(coeff * (dist - offs) ** 2)
    rf = jnp.concatenate([ea[:, i:i + 1] * df for i in range(EF)], axis=-1)
    er = jnp.concatenate([ea, rf], axis=-1)

    dot = functools.partial(jnp.dot, preferred_element_type=f32)
    w1e_k, w1d_k, w1s_k, b1_k, g_k, bt_k, w2_k, b2_k = wk
    w1e_v, w1d_v, w1s_v, b1_v, g_v, bt_v, w2_v, b2_v = wv

    bf = jnp.bfloat16
    erb = er.astype(bf)
    hdb = hd.astype(bf)
    hsb = hs.astype(bf)

    zk = dot(erb, w1e_k[...]) + dot(hdb, w1d_k[...]) + dot(hsb, w1s_k[...]) + b1_k[...]
    k = dot(_ln_relu(zk, g_k[...], bt_k[...]).astype(bf), w2_k[...]) + b2_k[...]

    zv = dot(erb, w1e_v[...]) + dot(hdb, w1d_v[...]) + dot(hsb, w1s_v[...]) + b1_v[...]
    v = dot(_ln_relu(zv, g_v[...], bt_v[...]).astype(bf), w2_v[...]) + b2_v[...]

    ew = jax.nn.sigmoid(jnp.sum(rf * eww[...], axis=-1, keepdims=True) + ewb[...])
    vsc = v * ew

    row = jax.lax.broadcasted_iota(jnp.int32, (D, H), 0) // DH
    col = jax.lax.broadcasted_iota(jnp.int32, (D, H), 1)
    sel = (row == col).astype(f32)
    logits = dot(qd * k, sel) * (1.0 / np.sqrt(DH))
    ex = jnp.exp(logits)
    return ex, vsc, rel


def _edge_body_x2h(ea_ref, gd_ref, gs_ref, dmod_ref, *refs):
    wk = refs[0:8]
    wv = refs[8:16]
    eww, ewb, exv_ref, den_ref = refs[16], refs[17], refs[18], refs[19]
    f32 = jnp.float32
    ex, vsc, _ = _edge_common(ea_ref[...], gd_ref[...], gs_ref[...], wk, wv, eww, ewb)

    row = jax.lax.broadcasted_iota(jnp.int32, (H, D), 0)
    col = jax.lax.broadcasted_iota(jnp.int32, (H, D), 1) // DH
    selt = (row == col).astype(f32)
    exv_ref[...] = jnp.dot(ex, selt, preferred_element_type=f32) * vsc

    dmod = dmod_ref[...]
    zer = jnp.zeros_like(ex)
    den = jnp.zeros((ex.shape[0], D), f32)
    for kk in range(8):
        row_k = jnp.concatenate([zer] * kk + [ex] + [zer] * (7 - kk), axis=-1)
        den = den + jnp.where(dmod == float(kk), row_k, 0.0)
    den_ref[...] = den


def _edge_body_h2x(ea_ref, gd_ref, gs_ref, *refs):
    wk = refs[0:8]
    wv = refs[8:16]
    eww, ewb, rows_ref = refs[16], refs[17], refs[18]
    ex, vsc, rel = _edge_common(ea_ref[...], gd_ref[...], gs_ref[...], wk, wv, eww, ewb)
    t = ex * vsc
    pieces = [t[:, hh:hh + 1] * rel for hh in range(H)]
    pad = jnp.zeros((ex.shape[0], D - 3 * H - H), jnp.float32)
    rows_ref[...] = jnp.concatenate(pieces + [ex, pad], axis=-1)


def _wsplit(p, dst_is_first):
    bf = jnp.bfloat16
    w1 = p["W1"]
    wa = w1[EF + RF:EF + RF + D]
    wb = w1[EF + RF + D:]
    wd, ws = (wa, wb) if dst_is_first else (wb, wa)
    return (w1[0:EF + RF].astype(bf), wd.astype(bf), ws.astype(bf),
            p["b1"].reshape(1, D), p["g"].reshape(1, D),
            p["bt"].reshape(1, D), p["W2"].astype(bf), p["b2"].reshape(1, -1))


def _edge_pass_x2h(ea, gd, gs, dmod, pk, pv, ew_w, ew_b):
    f32 = jnp.float32
    grid = E // TE

    def im_e(i):
        return (i, 0)

    def im_w(i):
        return (0, 0)

    weights = (*_wsplit(pk, True), *_wsplit(pv, True),
               ew_w.reshape(1, RF), ew_b.reshape(1, 1))
    espec = [pl.BlockSpec((TE, a.shape[1]), im_e) for a in (ea, gd, gs, dmod)]
    wspec = [pl.BlockSpec(w.shape, im_w) for w in weights]
    return pl.pallas_call(
        _edge_body_x2h,
        grid=(grid,),
        in_specs=espec + wspec,
        out_specs=[pl.BlockSpec((TE, D), im_e), pl.BlockSpec((TE, D), im_e)],
        out_shape=[jax.ShapeDtypeStruct((E, D), f32),
                   jax.ShapeDtypeStruct((E, D), f32)],
    )(ea, gd, gs, dmod, *weights)


def _edge_pass_h2x(ea, gd, gs, pk, pv, ew_w, ew_b):
    f32 = jnp.float32
    grid = E // TE

    def im_e(i):
        return (i, 0)

    def im_w(i):
        return (0, 0)

    weights = (*_wsplit(pk, False), *_wsplit(pv, False),
               ew_w.reshape(1, RF), ew_b.reshape(1, 1))
    espec = [pl.BlockSpec((TE, a.shape[1]), im_e) for a in (ea, gd, gs)]
    wspec = [pl.BlockSpec(w.shape, im_w) for w in weights]
    return pl.pallas_call(
        _edge_body_h2x,
        grid=(grid,),
        in_specs=espec + wspec,
        out_specs=pl.BlockSpec((TE, D), im_e),
        out_shape=jax.ShapeDtypeStruct((E, D), f32),
    )(ea, gd, gs, *weights)


def _mlp(p, x):
    hh = x @ p["W1"] + p["b1"]
    mu = jnp.mean(hh, axis=-1, keepdims=True)
    var = jnp.mean((hh - mu) ** 2, axis=-1, keepdims=True)
    hh = (hh - mu) / jnp.sqrt(var + 1e-5) * p["g"] + p["bt"]
    hh = jax.nn.relu(hh)
    return hh @ p["W2"] + p["b2"]


def kernel(h, x, edge_attr, mask_ligand, params, edge_index):
    f32 = jnp.float32
    dst = edge_index[1]
    src = edge_index[0]
    dst_r = dst.reshape(NW, NCH, GC)
    src_r = src.reshape(NW, NCH, GC)
    dden_r = (N + dst // 8).astype(jnp.int32).reshape(NW, NCH, GC)
    dmod = (dst % 8).astype(f32).reshape(E, 1)
    xpad = jnp.pad(x, ((0, 0), (0, XP - 3)))
    zeros1 = jnp.zeros((NACC1 // 16, D), f32)
    zeros2 = jnp.zeros((NACC2 // 16, D), f32)

    p = params["x2h"]
    q1 = _mlp(p["hq"], h)
    td = jnp.concatenate([h, q1, xpad], axis=1)
    ts = jnp.concatenate([h, xpad], axis=1)
    gd, gs = _sc_gather(td, ts, dst_r, src_r)
    exv, denrow = _edge_pass_x2h(edge_attr, gd, gs, dmod,
                                 p["hk"], p["hv"], p["ew_W"], p["ew_b"])
    part = _scatter1(exv, dst_r, denrow, dden_r, zeros1)
    tot = part[0] + part[1]
    num = tot[:N]
    den = tot[N:N + N // 8].reshape(N, H)
    out = (num.reshape(N, H, DH) / (den[:, :, None] + 1e-16)).reshape(N, D)
    out = _mlp(p["node"], jnp.concatenate([out, h], axis=-1))
    h_out = out + h

    p2 = params["h2x"]
    q2 = _mlp(p2["xq"], h_out)
    td2 = jnp.concatenate([h_out, q2, xpad], axis=1)
    ts2 = jnp.concatenate([h_out, xpad], axis=1)
    gd2, gs2 = _sc_gather(td2, ts2, dst_r, src_r)
    rows2 = _edge_pass_h2x(edge_attr, gd2, gs2,
                           p2["xk"], p2["xv"], p2["ew_W"], p2["ew_b"])
    part2 = _scatter2(rows2, dst_r, zeros2)
    tot2 = part2[0] + part2[1]
    num2 = tot2[:N, :3 * H].reshape(N, H, 3)
    den2 = tot2[:N, 3 * H:4 * H]
    delta = jnp.mean(num2 / (den2[:, :, None] + 1e-16), axis=1)
    x_out = x + delta * mask_ligand[:, None]
    return h_out, x_out


# speedup vs baseline: 1.0405x; 1.0405x over previous
import functools

import numpy as np
import jax
from jax import lax
import jax.numpy as jnp
from jax.experimental import pallas as pl
from jax.experimental.pallas import tpu as pltpu
from jax.experimental.pallas import tpu_sc as plsc

N = 10000
E = 320000
D = 128
H = 16
EF = 4
G = 20
RF = 80
DH = D // H
TE = 512
XP = 128
TDW = 2 * D + XP
TSW = D + XP

NW = 32
EPW = E // NW
GC = 80
NCH = EPW // GC

NACC1 = 11264
NACC2 = 10240

@functools.cache
def _sc_mesh():
    return plsc.VectorSubcoreMesh(core_axis_name="c", subcore_axis_name="s")


def _gather_body(td_hbm, ts_hbm, dsti_hbm, srci_hbm, gd_hbm, gs_hbm,
                 idx_d, idx_s, rows_d, rows_s, sem):
    wid = lax.axis_index("s") * 2 + lax.axis_index("c")
    pltpu.sync_copy(dsti_hbm.at[wid], idx_d)
    pltpu.sync_copy(srci_hbm.at[wid], idx_s)

    def chunk(j, carry):
        base = wid * EPW + j * GC
        cp1 = pltpu.async_copy(td_hbm.at[idx_d.at[j]], rows_d, sem)
        cp2 = pltpu.async_copy(ts_hbm.at[idx_s.at[j]], rows_s, sem)
        cp1.wait()
        cp2.wait()
        pltpu.sync_copy(rows_d, gd_hbm.at[pl.ds(base, GC)])
        pltpu.sync_copy(rows_s, gs_hbm.at[pl.ds(base, GC)])
        return carry

    lax.fori_loop(0, NCH, chunk, 0)


@functools.cache
def _build_sc_gather(tdw, tsw):
    return functools.partial(
        pl.kernel, _gather_body,
        mesh=_sc_mesh(),
        out_type=(jax.ShapeDtypeStruct((E, tdw), jnp.float32),
                  jax.ShapeDtypeStruct((E, tsw), jnp.float32)),
        scratch_types=[
            pltpu.VMEM((NCH, GC), jnp.int32),
            pltpu.VMEM((NCH, GC), jnp.int32),
            pltpu.VMEM((GC, tdw), jnp.float32),
            pltpu.VMEM((GC, tsw), jnp.float32),
            pltpu.SemaphoreType.DMA,
        ],
    )()


def _sc_gather(td, ts, dst_r, src_r):
    return _build_sc_gather(td.shape[1], ts.shape[1])(td, ts, dst_r, src_r)


def _make_scatter(nacc, nstreams):
    rpw = nacc // 16

    def body(*refs):
        ins = refs[:2 * nstreams]
        zeros_hbm = refs[2 * nstreams]
        out_hbm = refs[2 * nstreams + 1]
        scr = refs[2 * nstreams + 2:]
        idx = scr[0]
        rows = scr[1]
        acc = scr[2]

        cid = lax.axis_index("c")
        sid = lax.axis_index("s")
        wid = sid * 2 + cid
        pltpu.sync_copy(zeros_hbm, acc.at[pl.ds(sid * rpw, rpw)])
        plsc.subcore_barrier()

        def chunk(j, carry):
            base = wid * EPW + j * GC
            for t in range(nstreams):
                pltpu.sync_copy(ins[2 * t + 1].at[wid, j], idx)
                pltpu.sync_copy(ins[2 * t].at[pl.ds(base, GC)], rows)
                pltpu.sync_copy(rows, acc.at[idx], add=True)
            return carry

        lax.fori_loop(0, NCH, chunk, 0)
        plsc.subcore_barrier()
        pltpu.sync_copy(acc.at[pl.ds(sid * rpw, rpw)],
                        out_hbm.at[cid, pl.ds(sid * rpw, rpw)])

    scratch = [pltpu.VMEM((GC,), jnp.int32),
               pltpu.VMEM((GC, 128), jnp.float32),
               pltpu.VMEM_SHARED((nacc, 128), jnp.float32)]
    return functools.partial(
        pl.kernel, body,
        mesh=_sc_mesh(),
        out_type=jax.ShapeDtypeStruct((2, nacc, 128), jnp.float32),
        scratch_types=scratch,
    )()


_make_scatter = functools.cache(_make_scatter)


def _scatter1(*args):
    return _make_scatter(NACC1, 2)(*args)


def _scatter2(*args):
    return _make_scatter(NACC2, 1)(*args)


def _ln_relu(z, g, bt):
    mu = jnp.mean(z, axis=-1, keepdims=True)
    zc = z - mu
    var = jnp.mean(zc * zc, axis=-1, keepdims=True)
    zn = zc * jax.lax.rsqrt(var + 1e-5) * g + bt
    return jnp.maximum(zn, 0.0)


def _edge_common(ea, hd, qd, hs, rel, dist, wk, wv, eww, ewb):
    f32 = jnp.float32
    step = 10.0 / (G - 1)
    coeff = -0.5 / step ** 2
    offs = jax.lax.broadcasted_iota(jnp.int32, (1, G), 1).astype(f32) * step
    df = jnp.exp(coeff * (dist - offs) ** 2)
    rf = jnp.concatenate([ea[:, i:i + 1] * df for i in range(EF)], axis=-1)
    er = jnp.concatenate([ea, rf], axis=-1)

    dot = functools.partial(jnp.dot, preferred_element_type=f32)
    w1e_k, w1d_k, w1s_k, b1_k, g_k, bt_k, w2_k, b2_k = wk
    w1e_v, w1d_v, w1s_v, b1_v, g_v, bt_v, w2_v, b2_v = wv

    bf = jnp.bfloat16
    erb = er.astype(bf)
    hdb = hd.astype(bf)
    hsb = hs.astype(bf)

    zk = dot(erb, w1e_k[...]) + dot(hdb, w1d_k[...]) + dot(hsb, w1s_k[...]) + b1_k[...]
    k = dot(_ln_relu(zk, g_k[...], bt_k[...]).astype(bf), w2_k[...]) + b2_k[...]

    zv = dot(erb, w1e_v[...]) + dot(hdb, w1d_v[...]) + dot(hsb, w1s_v[...]) + b1_v[...]
    v = dot(_ln_relu(zv, g_v[...], bt_v[...]).astype(bf), w2_v[...]) + b2_v[...]

    ew = jax.nn.sigmoid(jnp.sum(rf * eww[...], axis=-1, keepdims=True) + ewb[...])
    vsc = v * ew

    row = jax.lax.broadcasted_iota(jnp.int32, (D, H), 0) // DH
    col = jax.lax.broadcasted_iota(jnp.int32, (D, H), 1)
    sel = (row == col).astype(f32)
    logits = dot(qd * k, sel) * (1.0 / np.sqrt(DH))
    ex = jnp.exp(logits)
    return ex, vsc


def _edge_body_x2h(ea_ref, gd_ref, gs_ref, dmod_ref, *refs):
    wk = refs[0:8]
    wv = refs[8:16]
    eww, ewb = refs[16], refs[17]
    exv_ref, den_ref, relp_ref = refs[18], refs[19], refs[20]
    f32 = jnp.float32
    gd = gd_ref[...]
    gs = gs_ref[...]
    rel = gd[:, 2 * D:2 * D + 3] - gs[:, D:D + 3]
    dist = jnp.sqrt(jnp.sum(rel * rel, axis=-1, keepdims=True) + 1e-12)
    relp_ref[...] = jnp.concatenate(
        [rel, dist, jnp.zeros((rel.shape[0], 4), f32)], axis=-1)
    ex, vsc = _edge_common(ea_ref[...], gd[:, 0:D], gd[:, D:2 * D],
                           gs[:, 0:D], rel, dist, wk, wv, eww, ewb)

    row = jax.lax.broadcasted_iota(jnp.int32, (H, D), 0)
    col = jax.lax.broadcasted_iota(jnp.int32, (H, D), 1) // DH
    selt = (row == col).astype(f32)
    exv_ref[...] = jnp.dot(ex, selt, preferred_element_type=f32) * vsc

    dmod = dmod_ref[...]
    zer = jnp.zeros_like(ex)
    den = jnp.zeros((ex.shape[0], D), f32)
    for kk in range(8):
        row_k = jnp.concatenate([zer] * kk + [ex] + [zer] * (7 - kk), axis=-1)
        den = den + jnp.where(dmod == float(kk), row_k, 0.0)
    den_ref[...] = den


def _edge_body_h2x(ea_ref, gd_ref, gs_ref, relp_ref, *refs):
    wk = refs[0:8]
    wv = refs[8:16]
    eww, ewb, rows_ref = refs[16], refs[17], refs[18]
    gd = gd_ref[...]
    relp = relp_ref[...]
    rel = relp[:, 0:3]
    dist = relp[:, 3:4]
    ex, vsc = _edge_common(ea_ref[...], gd[:, 0:D], gd[:, D:2 * D],
                           gs_ref[...], rel, dist, wk, wv, eww, ewb)
    t = ex * vsc
    pieces = [t[:, hh:hh + 1] * rel for hh in range(H)]
    pad = jnp.zeros((ex.shape[0], D - 3 * H - H), jnp.float32)
    rows_ref[...] = jnp.concatenate(pieces + [ex, pad], axis=-1)


def _wsplit(p, dst_is_first):
    bf = jnp.bfloat16
    w1 = p["W1"]
    wa = w1[EF + RF:EF + RF + D]
    wb = w1[EF + RF + D:]
    wd, ws = (wa, wb) if dst_is_first else (wb, wa)
    return (w1[0:EF + RF].astype(bf), wd.astype(bf), ws.astype(bf),
            p["b1"].reshape(1, D), p["g"].reshape(1, D),
            p["bt"].reshape(1, D), p["W2"].astype(bf), p["b2"].reshape(1, -1))


def _edge_pass_x2h(ea, gd, gs, dmod, pk, pv, ew_w, ew_b):
    f32 = jnp.float32
    grid = E // TE

    def im_e(i):
        return (i, 0)

    def im_w(i):
        return (0, 0)

    weights = (*_wsplit(pk, True), *_wsplit(pv, True),
               ew_w.reshape(1, RF), ew_b.reshape(1, 1))
    espec = [pl.BlockSpec((TE, a.shape[1]), im_e) for a in (ea, gd, gs, dmod)]
    wspec = [pl.BlockSpec(w.shape, im_w) for w in weights]
    return pl.pallas_call(
        _edge_body_x2h,
        grid=(grid,),
        in_specs=espec + wspec,
        out_specs=[pl.BlockSpec((TE, D), im_e), pl.BlockSpec((TE, D), im_e),
                   pl.BlockSpec((TE, 8), im_e)],
        out_shape=[jax.ShapeDtypeStruct((E, D), f32),
                   jax.ShapeDtypeStruct((E, D), f32),
                   jax.ShapeDtypeStruct((E, 8), f32)],
    )(ea, gd, gs, dmod, *weights)


def _edge_pass_h2x(ea, gd, gs, relp, pk, pv, ew_w, ew_b):
    f32 = jnp.float32
    grid = E // TE

    def im_e(i):
        return (i, 0)

    def im_w(i):
        return (0, 0)

    weights = (*_wsplit(pk, False), *_wsplit(pv, False),
               ew_w.reshape(1, RF), ew_b.reshape(1, 1))
    espec = [pl.BlockSpec((TE, a.shape[1]), im_e) for a in (ea, gd, gs, relp)]
    wspec = [pl.BlockSpec(w.shape, im_w) for w in weights]
    return pl.pallas_call(
        _edge_body_h2x,
        grid=(grid,),
        in_specs=espec + wspec,
        out_specs=pl.BlockSpec((TE, D), im_e),
        out_shape=jax.ShapeDtypeStruct((E, D), f32),
    )(ea, gd, gs, relp, *weights)


def _mlp(p, x):
    hh = x @ p["W1"] + p["b1"]
    mu = jnp.mean(hh, axis=-1, keepdims=True)
    var = jnp.mean((hh - mu) ** 2, axis=-1, keepdims=True)
    hh = (hh - mu) / jnp.sqrt(var + 1e-5) * p["g"] + p["bt"]
    hh = jax.nn.relu(hh)
    return hh @ p["W2"] + p["b2"]


def kernel(h, x, edge_attr, mask_ligand, params, edge_index):
    f32 = jnp.float32
    dst = edge_index[1]
    src = edge_index[0]
    dst_r = dst.reshape(NW, NCH, GC)
    src_r = src.reshape(NW, NCH, GC)
    dden_r = (N + dst // 8).astype(jnp.int32).reshape(NW, NCH, GC)
    dmod = (dst % 8).astype(f32).reshape(E, 1)
    xpad = jnp.pad(x, ((0, 0), (0, XP - 3)))
    zeros1 = jnp.zeros((NACC1 // 16, D), f32)
    zeros2 = jnp.zeros((NACC2 // 16, D), f32)

    p = params["x2h"]
    q1 = _mlp(p["hq"], h)
    td = jnp.concatenate([h, q1, xpad], axis=1)
    ts = jnp.concatenate([h, xpad], axis=1)
    gd, gs = _sc_gather(td, ts, dst_r, src_r)
    exv, denrow, relp = _edge_pass_x2h(edge_attr, gd, gs, dmod,
                                       p["hk"], p["hv"], p["ew_W"], p["ew_b"])
    part = _scatter1(exv, dst_r, denrow, dden_r, zeros1)
    tot = part[0] + part[1]
    num = tot[:N]
    den = tot[N:N + N // 8].reshape(N, H)
    out = (num.reshape(N, H, DH) / (den[:, :, None] + 1e-16)).reshape(N, D)
    out = _mlp(p["node"], jnp.concatenate([out, h], axis=-1))
    h_out = out + h

    p2 = params["h2x"]
    q2 = _mlp(p2["xq"], h_out)
    td2 = jnp.concatenate([h_out, q2], axis=1)
    ts2 = h_out
    gd2, gs2 = _sc_gather(td2, ts2, dst_r, src_r)
    rows2 = _edge_pass_h2x(edge_attr, gd2, gs2, relp,
                           p2["xk"], p2["xv"], p2["ew_W"], p2["ew_b"])
    part2 = _scatter2(rows2, dst_r, zeros2)
    tot2 = part2[0] + part2[1]
    num2 = tot2[:N, :3 * H].reshape(N, H, 3)
    den2 = tot2[:N, 3 * H:4 * H]
    delta = jnp.mean(num2 / (den2[:, :, None] + 1e-16), axis=1)
    x_out = x + delta * mask_ligand[:, None]
    return h_out, x_out


# speedup vs baseline: 1.0565x; 1.0154x over previous
import functools

import numpy as np
import jax
from jax import lax
import jax.numpy as jnp
from jax.experimental import pallas as pl
from jax.experimental.pallas import tpu as pltpu
from jax.experimental.pallas import tpu_sc as plsc

N = 10000
E = 320000
D = 128
H = 16
EF = 4
G = 20
RF = 80
DH = D // H
TE = 512
XP = 128
TDW = 2 * D + XP
TSW = D + XP

NW = 32
EPW = E // NW
GC = 80
NCH = EPW // GC
GCG = 40
NCHG = EPW // GCG

NACC1 = 11264
NACC2 = 10240

@functools.cache
def _sc_mesh():
    return plsc.VectorSubcoreMesh(core_axis_name="c", subcore_axis_name="s")


def _gather_body(td_hbm, ts_hbm, dsti_hbm, srci_hbm, gd_hbm, gs_hbm,
                 idx_d, idx_s, rows_d0, rows_s0, rows_d1, rows_s1,
                 sem0, sem1):
    wid = lax.axis_index("s") * 2 + lax.axis_index("c")
    pltpu.sync_copy(dsti_hbm.at[wid], idx_d)
    pltpu.sync_copy(srci_hbm.at[wid], idx_s)

    def start(j, rd, rs, sem):
        pltpu.async_copy(td_hbm.at[idx_d.at[j]], rd, sem)
        pltpu.async_copy(ts_hbm.at[idx_s.at[j]], rs, sem)

    def finish(j, rd, rs, sem):
        pltpu.make_async_copy(td_hbm.at[idx_d.at[j]], rd, sem).wait()
        pltpu.make_async_copy(ts_hbm.at[idx_s.at[j]], rs, sem).wait()
        base = wid * EPW + j * GCG
        pltpu.sync_copy(rd, gd_hbm.at[pl.ds(base, GCG)])
        pltpu.sync_copy(rs, gs_hbm.at[pl.ds(base, GCG)])

    start(0, rows_d0, rows_s0, sem0)

    def pair(i, carry):
        j0 = 2 * i
        start(j0 + 1, rows_d1, rows_s1, sem1)
        finish(j0, rows_d0, rows_s0, sem0)
        start(jnp.minimum(j0 + 2, NCHG - 2), rows_d0, rows_s0, sem0)
        finish(j0 + 1, rows_d1, rows_s1, sem1)
        return carry

    lax.fori_loop(0, NCHG // 2, pair, 0)
    pltpu.make_async_copy(td_hbm.at[idx_d.at[NCHG - 2]], rows_d0, sem0).wait()
    pltpu.make_async_copy(ts_hbm.at[idx_s.at[NCHG - 2]], rows_s0, sem0).wait()


@functools.cache
def _build_sc_gather(tdw, tsw):
    return functools.partial(
        pl.kernel, _gather_body,
        mesh=_sc_mesh(),
        out_type=(jax.ShapeDtypeStruct((E, tdw), jnp.float32),
                  jax.ShapeDtypeStruct((E, tsw), jnp.float32)),
        scratch_types=[
            pltpu.VMEM((NCHG, GCG), jnp.int32),
            pltpu.VMEM((NCHG, GCG), jnp.int32),
            pltpu.VMEM((GCG, tdw), jnp.float32),
            pltpu.VMEM((GCG, tsw), jnp.float32),
            pltpu.VMEM((GCG, tdw), jnp.float32),
            pltpu.VMEM((GCG, tsw), jnp.float32),
            pltpu.SemaphoreType.DMA,
            pltpu.SemaphoreType.DMA,
        ],
    )()


def _sc_gather(td, ts, dst_r, src_r):
    return _build_sc_gather(td.shape[1], ts.shape[1])(td, ts, dst_r, src_r)


def _make_scatter(nacc, nstreams):
    rpw = nacc // 16

    def body(*refs):
        ins = refs[:2 * nstreams]
        zeros_hbm = refs[2 * nstreams]
        out_hbm = refs[2 * nstreams + 1]
        scr = refs[2 * nstreams + 2:]
        idx = scr[0]
        rows = scr[1]
        acc = scr[2]

        cid = lax.axis_index("c")
        sid = lax.axis_index("s")
        wid = sid * 2 + cid
        pltpu.sync_copy(zeros_hbm, acc.at[pl.ds(sid * rpw, rpw)])
        plsc.subcore_barrier()

        def chunk(j, carry):
            base = wid * EPW + j * GC
            for t in range(nstreams):
                pltpu.sync_copy(ins[2 * t + 1].at[wid, j], idx)
                pltpu.sync_copy(ins[2 * t].at[pl.ds(base, GC)], rows)
                pltpu.sync_copy(rows, acc.at[idx], add=True)
            return carry

        lax.fori_loop(0, NCH, chunk, 0)
        plsc.subcore_barrier()
        pltpu.sync_copy(acc.at[pl.ds(sid * rpw, rpw)],
                        out_hbm.at[cid, pl.ds(sid * rpw, rpw)])

    scratch = [pltpu.VMEM((GC,), jnp.int32),
               pltpu.VMEM((GC, 128), jnp.float32),
               pltpu.VMEM_SHARED((nacc, 128), jnp.float32)]
    return functools.partial(
        pl.kernel, body,
        mesh=_sc_mesh(),
        out_type=jax.ShapeDtypeStruct((2, nacc, 128), jnp.float32),
        scratch_types=scratch,
    )()


_make_scatter = functools.cache(_make_scatter)


def _scatter1(*args):
    return _make_scatter(NACC1, 2)(*args)


def _scatter2(*args):
    return _make_scatter(NACC2, 1)(*args)


def _ln_relu(z, g, bt):
    mu = jnp.mean(z, axis=-1, keepdims=True)
    zc = z - mu
    var = jnp.mean(zc * zc, axis=-1, keepdims=True)
    zn = zc * jax.lax.rsqrt(var + 1e-5) * g + bt
    return jnp.maximum(zn, 0.0)


def _edge_common(ea, hd, qd, hs, rel, dist, wk, wv, eww, ewb):
    f32 = jnp.float32
    step = 10.0 / (G - 1)
    coeff = -0.5 / step ** 2
    offs = jax.lax.broadcasted_iota(jnp.int32, (1, G), 1).astype(f32) * step
    df = jnp.exp(coeff * (dist - offs) ** 2)
    rf = jnp.concatenate([ea[:, i:i + 1] * df for i in range(EF)], axis=-1)
    er = jnp.concatenate([ea, rf], axis=-1)

    dot = functools.partial(jnp.dot, preferred_element_type=f32)
    w1e_k, w1d_k, w1s_k, b1_k, g_k, bt_k, w2_k, b2_k = wk
    w1e_v, w1d_v, w1s_v, b1_v, g_v, bt_v, w2_v, b2_v = wv

    bf = jnp.bfloat16
    erb = er.astype(bf)
    hdb = hd.astype(bf)
    hsb = hs.astype(bf)

    zk = dot(erb, w1e_k[...]) + dot(hdb, w1d_k[...]) + dot(hsb, w1s_k[...]) + b1_k[...]
    k = dot(_ln_relu(zk, g_k[...], bt_k[...]).astype(bf), w2_k[...]) + b2_k[...]

    zv = dot(erb, w1e_v[...]) + dot(hdb, w1d_v[...]) + dot(hsb, w1s_v[...]) + b1_v[...]
    v = dot(_ln_relu(zv, g_v[...], bt_v[...]).astype(bf), w2_v[...]) + b2_v[...]

    ew = jax.nn.sigmoid(jnp.sum(rf * eww[...], axis=-1, keepdims=True) + ewb[...])
    vsc = v * ew

    row = jax.lax.broadcasted_iota(jnp.int32, (D, H), 0) // DH
    col = jax.lax.broadcasted_iota(jnp.int32, (D, H), 1)
    sel = (row == col).astype(f32)
    logits = dot(qd * k, sel) * (1.0 / np.sqrt(DH))
    ex = jnp.exp(logits)
    return ex, vsc


def _edge_body_x2h(ea_ref, gd_ref, gs_ref, dmod_ref, *refs):
    wk = refs[0:8]
    wv = refs[8:16]
    eww, ewb = refs[16], refs[17]
    exv_ref, den_ref, relp_ref = refs[18], refs[19], refs[20]
    f32 = jnp.float32
    gd = gd_ref[...]
    gs = gs_ref[...]
    rel = gd[:, 2 * D:2 * D + 3] - gs[:, D:D + 3]
    dist = jnp.sqrt(jnp.sum(rel * rel, axis=-1, keepdims=True) + 1e-12)
    relp_ref[...] = jnp.concatenate(
        [rel, dist, jnp.zeros((rel.shape[0], 4), f32)], axis=-1)
    ex, vsc = _edge_common(ea_ref[...], gd[:, 0:D], gd[:, D:2 * D],
                           gs[:, 0:D], rel, dist, wk, wv, eww, ewb)

    row = jax.lax.broadcasted_iota(jnp.int32, (H, D), 0)
    col = jax.lax.broadcasted_iota(jnp.int32, (H, D), 1) // DH
    selt = (row == col).astype(f32)
    exv_ref[...] = jnp.dot(ex, selt, preferred_element_type=f32) * vsc

    dmod = dmod_ref[...]
    zer = jnp.zeros_like(ex)
    den = jnp.zeros((ex.shape[0], D), f32)
    for kk in range(8):
        row_k = jnp.concatenate([zer] * kk + [ex] + [zer] * (7 - kk), axis=-1)
        den = den + jnp.where(dmod == float(kk), row_k, 0.0)
    den_ref[...] = den


def _edge_body_h2x(ea_ref, gd_ref, gs_ref, relp_ref, *refs):
    wk = refs[0:8]
    wv = refs[8:16]
    eww, ewb, rows_ref = refs[16], refs[17], refs[18]
    gd = gd_ref[...]
    relp = relp_ref[...]
    rel = relp[:, 0:3]
    dist = relp[:, 3:4]
    ex, vsc = _edge_common(ea_ref[...], gd[:, 0:D], gd[:, D:2 * D],
                           gs_ref[...], rel, dist, wk, wv, eww, ewb)
    t = ex * vsc
    pieces = [t[:, hh:hh + 1] * rel for hh in range(H)]
    pad = jnp.zeros((ex.shape[0], D - 3 * H - H), jnp.float32)
    rows_ref[...] = jnp.concatenate(pieces + [ex, pad], axis=-1)


def _wsplit(p, dst_is_first):
    bf = jnp.bfloat16
    w1 = p["W1"]
    wa = w1[EF + RF:EF + RF + D]
    wb = w1[EF + RF + D:]
    wd, ws = (wa, wb) if dst_is_first else (wb, wa)
    return (w1[0:EF + RF].astype(bf), wd.astype(bf), ws.astype(bf),
            p["b1"].reshape(1, D), p["g"].reshape(1, D),
            p["bt"].reshape(1, D), p["W2"].astype(bf), p["b2"].reshape(1, -1))


def _edge_pass_x2h(ea, gd, gs, dmod, pk, pv, ew_w, ew_b):
    f32 = jnp.float32
    grid = E // TE

    def im_e(i):
        return (i, 0)

    def im_w(i):
        return (0, 0)

    weights = (*_wsplit(pk, True), *_wsplit(pv, True),
               ew_w.reshape(1, RF), ew_b.reshape(1, 1))
    espec = [pl.BlockSpec((TE, a.shape[1]), im_e) for a in (ea, gd, gs, dmod)]
    wspec = [pl.BlockSpec(w.shape, im_w) for w in weights]
    return pl.pallas_call(
        _edge_body_x2h,
        grid=(grid,),
        in_specs=espec + wspec,
        out_specs=[pl.BlockSpec((TE, D), im_e), pl.BlockSpec((TE, D), im_e),
                   pl.BlockSpec((TE, 8), im_e)],
        out_shape=[jax.ShapeDtypeStruct((E, D), f32),
                   jax.ShapeDtypeStruct((E, D), f32),
                   jax.ShapeDtypeStruct((E, 8), f32)],
    )(ea, gd, gs, dmod, *weights)


def _edge_pass_h2x(ea, gd, gs, relp, pk, pv, ew_w, ew_b):
    f32 = jnp.float32
    grid = E // TE

    def im_e(i):
        return (i, 0)

    def im_w(i):
        return (0, 0)

    weights = (*_wsplit(pk, False), *_wsplit(pv, False),
               ew_w.reshape(1, RF), ew_b.reshape(1, 1))
    espec = [pl.BlockSpec((TE, a.shape[1]), im_e) for a in (ea, gd, gs, relp)]
    wspec = [pl.BlockSpec(w.shape, im_w) for w in weights]
    return pl.pallas_call(
        _edge_body_h2x,
        grid=(grid,),
        in_specs=espec + wspec,
        out_specs=pl.BlockSpec((TE, D), im_e),
        out_shape=jax.ShapeDtypeStruct((E, D), f32),
    )(ea, gd, gs, relp, *weights)


def _mlp(p, x):
    hh = x @ p["W1"] + p["b1"]
    mu = jnp.mean(hh, axis=-1, keepdims=True)
    var = jnp.mean((hh - mu) ** 2, axis=-1, keepdims=True)
    hh = (hh - mu) / jnp.sqrt(var + 1e-5) * p["g"] + p["bt"]
    hh = jax.nn.relu(hh)
    return hh @ p["W2"] + p["b2"]


def kernel(h, x, edge_attr, mask_ligand, params, edge_index):
    f32 = jnp.float32
    dst = edge_index[1]
    src = edge_index[0]
    dst_r = dst.reshape(NW, NCH, GC)
    src_r = src.reshape(NW, NCH, GC)
    dst_g = dst.reshape(NW, NCHG, GCG)
    src_g = src.reshape(NW, NCHG, GCG)
    dden_r = (N + dst // 8).astype(jnp.int32).reshape(NW, NCH, GC)
    dmod = (dst % 8).astype(f32).reshape(E, 1)
    xpad = jnp.pad(x, ((0, 0), (0, XP - 3)))
    zeros1 = jnp.zeros((NACC1 // 16, D), f32)
    zeros2 = jnp.zeros((NACC2 // 16, D), f32)

    p = params["x2h"]
    q1 = _mlp(p["hq"], h)
    td = jnp.concatenate([h, q1, xpad], axis=1)
    ts = jnp.concatenate([h, xpad], axis=1)
    gd, gs = _sc_gather(td, ts, dst_g, src_g)
    exv, denrow, relp = _edge_pass_x2h(edge_attr, gd, gs, dmod,
                                       p["hk"], p["hv"], p["ew_W"], p["ew_b"])
    part = _scatter1(exv, dst_r, denrow, dden_r, zeros1)
    tot = part[0] + part[1]
    num = tot[:N]
    den = tot[N:N + N // 8].reshape(N, H)
    out = (num.reshape(N, H, DH) / (den[:, :, None] + 1e-16)).reshape(N, D)
    out = _mlp(p["node"], jnp.concatenate([out, h], axis=-1))
    h_out = out + h

    p2 = params["h2x"]
    q2 = _mlp(p2["xq"], h_out)
    td2 = jnp.concatenate([h_out, q2], axis=1)
    ts2 = h_out
    gd2, gs2 = _sc_gather(td2, ts2, dst_g, src_g)
    rows2 = _edge_pass_h2x(edge_attr, gd2, gs2, relp,
                           p2["xk"], p2["xv"], p2["ew_W"], p2["ew_b"])
    part2 = _scatter2(rows2, dst_r, zeros2)
    tot2 = part2[0] + part2[1]
    num2 = tot2[:N, :3 * H].reshape(N, H, 3)
    den2 = tot2[:N, 3 * H:4 * H]
    delta = jnp.mean(num2 / (den2[:, :, None] + 1e-16), axis=1)
    x_out = x + delta * mask_ligand[:, None]
    return h_out, x_out


# speedup vs baseline: 1.1339x; 1.0733x over previous
import functools

import numpy as np
import jax
from jax import lax
import jax.numpy as jnp
from jax.experimental import pallas as pl
from jax.experimental.pallas import tpu as pltpu
from jax.experimental.pallas import tpu_sc as plsc

N = 10000
E = 320000
D = 128
H = 16
EF = 4
G = 20
RF = 80
DH = D // H
TE = 1280
XP = 128
TDW = 2 * D + XP
TSW = D + XP

NW = 32
EPW = E // NW
GC = 80
NCH = EPW // GC
GCG = 40
NCHG = EPW // GCG

NACC1 = 11264
NACC2 = 10240

@functools.cache
def _sc_mesh():
    return plsc.VectorSubcoreMesh(core_axis_name="c", subcore_axis_name="s")


def _gather_body(td_hbm, ts_hbm, dsti_hbm, srci_hbm, gd_hbm, gs_hbm,
                 idx_d, idx_s, rows_d0, rows_s0, rows_d1, rows_s1,
                 sem0, sem1):
    wid = lax.axis_index("s") * 2 + lax.axis_index("c")
    pltpu.sync_copy(dsti_hbm.at[wid], idx_d)
    pltpu.sync_copy(srci_hbm.at[wid], idx_s)

    def start(j, rd, rs, sem):
        pltpu.async_copy(td_hbm.at[idx_d.at[j]], rd, sem)
        pltpu.async_copy(ts_hbm.at[idx_s.at[j]], rs, sem)

    def finish(j, rd, rs, sem):
        pltpu.make_async_copy(td_hbm.at[idx_d.at[j]], rd, sem).wait()
        pltpu.make_async_copy(ts_hbm.at[idx_s.at[j]], rs, sem).wait()
        base = wid * EPW + j * GCG
        pltpu.sync_copy(rd, gd_hbm.at[pl.ds(base, GCG)])
        pltpu.sync_copy(rs, gs_hbm.at[pl.ds(base, GCG)])

    start(0, rows_d0, rows_s0, sem0)

    def pair(i, carry):
        j0 = 2 * i
        start(j0 + 1, rows_d1, rows_s1, sem1)
        finish(j0, rows_d0, rows_s0, sem0)
        start(jnp.minimum(j0 + 2, NCHG - 2), rows_d0, rows_s0, sem0)
        finish(j0 + 1, rows_d1, rows_s1, sem1)
        return carry

    lax.fori_loop(0, NCHG // 2, pair, 0)
    pltpu.make_async_copy(td_hbm.at[idx_d.at[NCHG - 2]], rows_d0, sem0).wait()
    pltpu.make_async_copy(ts_hbm.at[idx_s.at[NCHG - 2]], rows_s0, sem0).wait()


@functools.cache
def _build_sc_gather(tdw, tsw):
    return functools.partial(
        pl.kernel, _gather_body,
        mesh=_sc_mesh(),
        out_type=(jax.ShapeDtypeStruct((E, tdw), jnp.float32),
                  jax.ShapeDtypeStruct((E, tsw), jnp.float32)),
        scratch_types=[
            pltpu.VMEM((NCHG, GCG), jnp.int32),
            pltpu.VMEM((NCHG, GCG), jnp.int32),
            pltpu.VMEM((GCG, tdw), jnp.float32),
            pltpu.VMEM((GCG, tsw), jnp.float32),
            pltpu.VMEM((GCG, tdw), jnp.float32),
            pltpu.VMEM((GCG, tsw), jnp.float32),
            pltpu.SemaphoreType.DMA,
            pltpu.SemaphoreType.DMA,
        ],
    )()


def _sc_gather(td, ts, dst_r, src_r):
    return _build_sc_gather(td.shape[1], ts.shape[1])(td, ts, dst_r, src_r)


def _make_scatter(nacc, nstreams):
    rpw = nacc // 16

    def body(*refs):
        ins = refs[:2 * nstreams]
        zeros_hbm = refs[2 * nstreams]
        out_hbm = refs[2 * nstreams + 1]
        scr = refs[2 * nstreams + 2:]
        idx = scr[0]
        rows = scr[1]
        acc = scr[2]

        cid = lax.axis_index("c")
        sid = lax.axis_index("s")
        wid = sid * 2 + cid
        pltpu.sync_copy(zeros_hbm, acc.at[pl.ds(sid * rpw, rpw)])
        plsc.subcore_barrier()

        def chunk(j, carry):
            base = wid * EPW + j * GC
            for t in range(nstreams):
                pltpu.sync_copy(ins[2 * t + 1].at[wid, j], idx)
                pltpu.sync_copy(ins[2 * t].at[pl.ds(base, GC)], rows)
                pltpu.sync_copy(rows, acc.at[idx], add=True)
            return carry

        lax.fori_loop(0, NCH, chunk, 0)
        plsc.subcore_barrier()
        pltpu.sync_copy(acc.at[pl.ds(sid * rpw, rpw)],
                        out_hbm.at[cid, pl.ds(sid * rpw, rpw)])

    scratch = [pltpu.VMEM((GC,), jnp.int32),
               pltpu.VMEM((GC, 128), jnp.float32),
               pltpu.VMEM_SHARED((nacc, 128), jnp.float32)]
    return functools.partial(
        pl.kernel, body,
        mesh=_sc_mesh(),
        out_type=jax.ShapeDtypeStruct((2, nacc, 128), jnp.float32),
        scratch_types=scratch,
    )()


_make_scatter = functools.cache(_make_scatter)


def _scatter1(*args):
    return _make_scatter(NACC1, 2)(*args)


def _scatter2(*args):
    return _make_scatter(NACC2, 1)(*args)


def _ln_relu(z, g, bt):
    mu = jnp.mean(z, axis=-1, keepdims=True)
    zc = z - mu
    var = jnp.mean(zc * zc, axis=-1, keepdims=True)
    zn = zc * jax.lax.rsqrt(var + 1e-5) * g + bt
    return jnp.maximum(zn, 0.0)


def _edge_common(ea, hd, qd, hs, rel, dist, wk, wv, eww, ewb):
    f32 = jnp.float32
    step = 10.0 / (G - 1)
    coeff = -0.5 / step ** 2
    offs = jax.lax.broadcasted_iota(jnp.int32, (1, G), 1).astype(f32) * step
    df = jnp.exp(coeff * (dist - offs) ** 2)
    rf = jnp.concatenate([ea[:, i:i + 1] * df for i in range(EF)], axis=-1)
    er = jnp.concatenate([ea, rf], axis=-1)

    dot = functools.partial(jnp.dot, preferred_element_type=f32)
    w1e_k, w1d_k, w1s_k, b1_k, g_k, bt_k, w2_k, b2_k = wk
    w1e_v, w1d_v, w1s_v, b1_v, g_v, bt_v, w2_v, b2_v = wv

    bf = jnp.bfloat16
    erb = er.astype(bf)
    hdb = hd.astype(bf)
    hsb = hs.astype(bf)

    zk = dot(erb, w1e_k[...]) + dot(hdb, w1d_k[...]) + dot(hsb, w1s_k[...]) + b1_k[...]
    k = dot(_ln_relu(zk, g_k[...], bt_k[...]).astype(bf), w2_k[...]) + b2_k[...]

    zv = dot(erb, w1e_v[...]) + dot(hdb, w1d_v[...]) + dot(hsb, w1s_v[...]) + b1_v[...]
    v = dot(_ln_relu(zv, g_v[...], bt_v[...]).astype(bf), w2_v[...]) + b2_v[...]

    ew = jax.nn.sigmoid(jnp.sum(rf * eww[...], axis=-1, keepdims=True) + ewb[...])
    vsc = v * ew

    row = jax.lax.broadcasted_iota(jnp.int32, (D, H), 0) // DH
    col = jax.lax.broadcasted_iota(jnp.int32, (D, H), 1)
    sel = (row == col).astype(f32)
    logits = dot(qd * k, sel) * (1.0 / np.sqrt(DH))
    ex = jnp.exp(logits)
    return ex, vsc


def _edge_body_x2h(ea_ref, gd_ref, gs_ref, dmod_ref, *refs):
    wk = refs[0:8]
    wv = refs[8:16]
    eww, ewb = refs[16], refs[17]
    exv_ref, den_ref, relp_ref = refs[18], refs[19], refs[20]
    f32 = jnp.float32
    gd = gd_ref[...]
    gs = gs_ref[...]
    rel = gd[:, 2 * D:2 * D + 3] - gs[:, D:D + 3]
    dist = jnp.sqrt(jnp.sum(rel * rel, axis=-1, keepdims=True) + 1e-12)
    relp_ref[...] = jnp.concatenate(
        [rel, dist, jnp.zeros((rel.shape[0], 4), f32)], axis=-1)
    ex, vsc = _edge_common(ea_ref[...], gd[:, 0:D], gd[:, D:2 * D],
                           gs[:, 0:D], rel, dist, wk, wv, eww, ewb)

    row = jax.lax.broadcasted_iota(jnp.int32, (H, D), 0)
    col = jax.lax.broadcasted_iota(jnp.int32, (H, D), 1) // DH
    selt = (row == col).astype(f32)
    exv_ref[...] = jnp.dot(ex, selt, preferred_element_type=f32) * vsc

    dmod = dmod_ref[...]
    zer = jnp.zeros_like(ex)
    den = jnp.zeros((ex.shape[0], D), f32)
    for kk in range(8):
        row_k = jnp.concatenate([zer] * kk + [ex] + [zer] * (7 - kk), axis=-1)
        den = den + jnp.where(dmod == float(kk), row_k, 0.0)
    den_ref[...] = den


def _edge_body_h2x(ea_ref, gd_ref, gs_ref, relp_ref, *refs):
    wk = refs[0:8]
    wv = refs[8:16]
    eww, ewb, rows_ref = refs[16], refs[17], refs[18]
    gd = gd_ref[...]
    relp = relp_ref[...]
    rel = relp[:, 0:3]
    dist = relp[:, 3:4]
    ex, vsc = _edge_common(ea_ref[...], gd[:, 0:D], gd[:, D:2 * D],
                           gs_ref[...], rel, dist, wk, wv, eww, ewb)
    t = ex * vsc
    pieces = [t[:, hh:hh + 1] * rel for hh in range(H)]
    pad = jnp.zeros((ex.shape[0], D - 3 * H - H), jnp.float32)
    rows_ref[...] = jnp.concatenate(pieces + [ex, pad], axis=-1)


def _wsplit(p, dst_is_first):
    bf = jnp.bfloat16
    w1 = p["W1"]
    wa = w1[EF + RF:EF + RF + D]
    wb = w1[EF + RF + D:]
    wd, ws = (wa, wb) if dst_is_first else (wb, wa)
    return (w1[0:EF + RF].astype(bf), wd.astype(bf), ws.astype(bf),
            p["b1"].reshape(1, D), p["g"].reshape(1, D),
            p["bt"].reshape(1, D), p["W2"].astype(bf), p["b2"].reshape(1, -1))


def _edge_pass_x2h(ea, gd, gs, dmod, pk, pv, ew_w, ew_b):
    f32 = jnp.float32
    grid = E // TE

    def im_e(i):
        return (i, 0)

    def im_w(i):
        return (0, 0)

    weights = (*_wsplit(pk, True), *_wsplit(pv, True),
               ew_w.reshape(1, RF), ew_b.reshape(1, 1))
    espec = [pl.BlockSpec((TE, a.shape[1]), im_e) for a in (ea, gd, gs, dmod)]
    wspec = [pl.BlockSpec(w.shape, im_w) for w in weights]
    return pl.pallas_call(
        _edge_body_x2h,
        grid=(grid,),
        in_specs=espec + wspec,
        out_specs=[pl.BlockSpec((TE, D), im_e), pl.BlockSpec((TE, D), im_e),
                   pl.BlockSpec((TE, 8), im_e)],
        out_shape=[jax.ShapeDtypeStruct((E, D), f32),
                   jax.ShapeDtypeStruct((E, D), f32),
                   jax.ShapeDtypeStruct((E, 8), f32)],
    )(ea, gd, gs, dmod, *weights)


def _edge_pass_h2x(ea, gd, gs, relp, pk, pv, ew_w, ew_b):
    f32 = jnp.float32
    grid = E // TE

    def im_e(i):
        return (i, 0)

    def im_w(i):
        return (0, 0)

    weights = (*_wsplit(pk, False), *_wsplit(pv, False),
               ew_w.reshape(1, RF), ew_b.reshape(1, 1))
    espec = [pl.BlockSpec((TE, a.shape[1]), im_e) for a in (ea, gd, gs, relp)]
    wspec = [pl.BlockSpec(w.shape, im_w) for w in weights]
    return pl.pallas_call(
        _edge_body_h2x,
        grid=(grid,),
        in_specs=espec + wspec,
        out_specs=pl.BlockSpec((TE, D), im_e),
        out_shape=jax.ShapeDtypeStruct((E, D), f32),
    )(ea, gd, gs, relp, *weights)


def _mlp(p, x):
    hh = x @ p["W1"] + p["b1"]
    mu = jnp.mean(hh, axis=-1, keepdims=True)
    var = jnp.mean((hh - mu) ** 2, axis=-1, keepdims=True)
    hh = (hh - mu) / jnp.sqrt(var + 1e-5) * p["g"] + p["bt"]
    hh = jax.nn.relu(hh)
    return hh @ p["W2"] + p["b2"]


def kernel(h, x, edge_attr, mask_ligand, params, edge_index):
    f32 = jnp.float32
    dst = edge_index[1]
    src = edge_index[0]
    dst_r = dst.reshape(NW, NCH, GC)
    src_r = src.reshape(NW, NCH, GC)
    dst_g = dst.reshape(NW, NCHG, GCG)
    src_g = src.reshape(NW, NCHG, GCG)
    dden_r = (N + dst // 8).astype(jnp.int32).reshape(NW, NCH, GC)
    dmod = (dst % 8).astype(f32).reshape(E, 1)
    xpad = jnp.pad(x, ((0, 0), (0, XP - 3)))
    zeros1 = jnp.zeros((NACC1 // 16, D), f32)
    zeros2 = jnp.zeros((NACC2 // 16, D), f32)

    p = params["x2h"]
    q1 = _mlp(p["hq"], h)
    td = jnp.concatenate([h, q1, xpad], axis=1)
    ts = jnp.concatenate([h, xpad], axis=1)
    gd, gs = _sc_gather(td, ts, dst_g, src_g)
    exv, denrow, relp = _edge_pass_x2h(edge_attr, gd, gs, dmod,
                                       p["hk"], p["hv"], p["ew_W"], p["ew_b"])
    part = _scatter1(exv, dst_r, denrow, dden_r, zeros1)
    tot = part[0] + part[1]
    num = tot[:N]
    den = tot[N:N + N // 8].reshape(N, H)
    out = (num.reshape(N, H, DH) / (den[:, :, None] + 1e-16)).reshape(N, D)
    out = _mlp(p["node"], jnp.concatenate([out, h], axis=-1))
    h_out = out + h

    p2 = params["h2x"]
    q2 = _mlp(p2["xq"], h_out)
    td2 = jnp.concatenate([h_out, q2], axis=1)
    ts2 = h_out
    gd2, gs2 = _sc_gather(td2, ts2, dst_g, src_g)
    rows2 = _edge_pass_h2x(edge_attr, gd2, gs2, relp,
                           p2["xk"], p2["xv"], p2["ew_W"], p2["ew_b"])
    part2 = _scatter2(rows2, dst_r, zeros2)
    tot2 = part2[0] + part2[1]
    num2 = tot2[:N, :3 * H].reshape(N, H, 3)
    den2 = tot2[:N, 3 * H:4 * H]
    delta = jnp.mean(num2 / (den2[:, :, None] + 1e-16), axis=1)
    x_out = x + delta * mask_ligand[:, None]
    return h_out, x_out


# speedup vs baseline: 1.1836x; 1.0438x over previous
import functools

import numpy as np
import jax
from jax import lax
import jax.numpy as jnp
from jax.experimental import pallas as pl
from jax.experimental.pallas import tpu as pltpu
from jax.experimental.pallas import tpu_sc as plsc

N = 10000
E = 320000
D = 128
H = 16
EF = 4
G = 20
RF = 80
DH = D // H
TE = 2560
XP = 128
TDW = 2 * D + XP
TSW = D + XP

NW = 32
EPW = E // NW
GC = 80
NCH = EPW // GC
GCG = 40
NCHG = EPW // GCG

NACC1 = 11264
NACC2 = 10240

@functools.cache
def _sc_mesh():
    return plsc.VectorSubcoreMesh(core_axis_name="c", subcore_axis_name="s")


def _gather_body(td_hbm, ts_hbm, dsti_hbm, srci_hbm, gd_hbm, gs_hbm,
                 idx_d, idx_s, rows_d0, rows_s0, rows_d1, rows_s1,
                 sem0, sem1):
    wid = lax.axis_index("s") * 2 + lax.axis_index("c")
    pltpu.sync_copy(dsti_hbm.at[wid], idx_d)
    pltpu.sync_copy(srci_hbm.at[wid], idx_s)

    def start(j, rd, rs, sem):
        pltpu.async_copy(td_hbm.at[idx_d.at[j]], rd, sem)
        pltpu.async_copy(ts_hbm.at[idx_s.at[j]], rs, sem)

    def finish(j, rd, rs, sem):
        pltpu.make_async_copy(td_hbm.at[idx_d.at[j]], rd, sem).wait()
        pltpu.make_async_copy(ts_hbm.at[idx_s.at[j]], rs, sem).wait()
        base = wid * EPW + j * GCG
        pltpu.sync_copy(rd, gd_hbm.at[pl.ds(base, GCG)])
        pltpu.sync_copy(rs, gs_hbm.at[pl.ds(base, GCG)])

    start(0, rows_d0, rows_s0, sem0)

    def pair(i, carry):
        j0 = 2 * i
        start(j0 + 1, rows_d1, rows_s1, sem1)
        finish(j0, rows_d0, rows_s0, sem0)
        start(jnp.minimum(j0 + 2, NCHG - 2), rows_d0, rows_s0, sem0)
        finish(j0 + 1, rows_d1, rows_s1, sem1)
        return carry

    lax.fori_loop(0, NCHG // 2, pair, 0)
    pltpu.make_async_copy(td_hbm.at[idx_d.at[NCHG - 2]], rows_d0, sem0).wait()
    pltpu.make_async_copy(ts_hbm.at[idx_s.at[NCHG - 2]], rows_s0, sem0).wait()


@functools.cache
def _build_sc_gather(tdw, tsw):
    return functools.partial(
        pl.kernel, _gather_body,
        mesh=_sc_mesh(),
        out_type=(jax.ShapeDtypeStruct((E, tdw), jnp.float32),
                  jax.ShapeDtypeStruct((E, tsw), jnp.float32)),
        scratch_types=[
            pltpu.VMEM((NCHG, GCG), jnp.int32),
            pltpu.VMEM((NCHG, GCG), jnp.int32),
            pltpu.VMEM((GCG, tdw), jnp.float32),
            pltpu.VMEM((GCG, tsw), jnp.float32),
            pltpu.VMEM((GCG, tdw), jnp.float32),
            pltpu.VMEM((GCG, tsw), jnp.float32),
            pltpu.SemaphoreType.DMA,
            pltpu.SemaphoreType.DMA,
        ],
    )()


def _sc_gather(td, ts, dst_r, src_r):
    return _build_sc_gather(td.shape[1], ts.shape[1])(td, ts, dst_r, src_r)


def _make_scatter(nacc, nstreams):
    rpw = nacc // 16

    def body(*refs):
        ins = refs[:2 * nstreams]
        zeros_hbm = refs[2 * nstreams]
        out_hbm = refs[2 * nstreams + 1]
        scr = refs[2 * nstreams + 2:]
        idx = scr[0]
        rows = scr[1]
        acc = scr[2]

        cid = lax.axis_index("c")
        sid = lax.axis_index("s")
        wid = sid * 2 + cid
        pltpu.sync_copy(zeros_hbm, acc.at[pl.ds(sid * rpw, rpw)])
        plsc.subcore_barrier()

        def chunk(j, carry):
            base = wid * EPW + j * GC
            for t in range(nstreams):
                pltpu.sync_copy(ins[2 * t + 1].at[wid, j], idx)
                pltpu.sync_copy(ins[2 * t].at[pl.ds(base, GC)], rows)
                pltpu.sync_copy(rows, acc.at[idx], add=True)
            return carry

        lax.fori_loop(0, NCH, chunk, 0)
        plsc.subcore_barrier()
        pltpu.sync_copy(acc.at[pl.ds(sid * rpw, rpw)],
                        out_hbm.at[cid, pl.ds(sid * rpw, rpw)])

    scratch = [pltpu.VMEM((GC,), jnp.int32),
               pltpu.VMEM((GC, 128), jnp.float32),
               pltpu.VMEM_SHARED((nacc, 128), jnp.float32)]
    return functools.partial(
        pl.kernel, body,
        mesh=_sc_mesh(),
        out_type=jax.ShapeDtypeStruct((2, nacc, 128), jnp.float32),
        scratch_types=scratch,
    )()


_make_scatter = functools.cache(_make_scatter)


def _scatter1(*args):
    return _make_scatter(NACC1, 2)(*args)


def _scatter2(*args):
    return _make_scatter(NACC2, 1)(*args)


def _ln_relu(z, g, bt):
    mu = jnp.mean(z, axis=-1, keepdims=True)
    zc = z - mu
    var = jnp.mean(zc * zc, axis=-1, keepdims=True)
    zn = zc * jax.lax.rsqrt(var + 1e-5) * g + bt
    return jnp.maximum(zn, 0.0)


def _edge_common(ea, hd, qd, hs, rel, dist, wk, wv, eww, ewb):
    f32 = jnp.float32
    step = 10.0 / (G - 1)
    coeff = -0.5 / step ** 2
    offs = jax.lax.broadcasted_iota(jnp.int32, (1, G), 1).astype(f32) * step
    df = jnp.exp(coeff * (dist - offs) ** 2)
    rf = jnp.concatenate([ea[:, i:i + 1] * df for i in range(EF)], axis=-1)
    er = jnp.concatenate([ea, rf], axis=-1)

    dot = functools.partial(jnp.dot, preferred_element_type=f32)
    w1e_k, w1d_k, w1s_k, b1_k, g_k, bt_k, w2_k, b2_k = wk
    w1e_v, w1d_v, w1s_v, b1_v, g_v, bt_v, w2_v, b2_v = wv

    bf = jnp.bfloat16
    erb = er.astype(bf)
    hdb = hd.astype(bf)
    hsb = hs.astype(bf)

    zk = dot(erb, w1e_k[...]) + dot(hdb, w1d_k[...]) + dot(hsb, w1s_k[...]) + b1_k[...]
    k = dot(_ln_relu(zk, g_k[...], bt_k[...]).astype(bf), w2_k[...]) + b2_k[...]

    zv = dot(erb, w1e_v[...]) + dot(hdb, w1d_v[...]) + dot(hsb, w1s_v[...]) + b1_v[...]
    v = dot(_ln_relu(zv, g_v[...], bt_v[...]).astype(bf), w2_v[...]) + b2_v[...]

    ew = jax.nn.sigmoid(jnp.sum(rf * eww[...], axis=-1, keepdims=True) + ewb[...])
    vsc = v * ew

    row = jax.lax.broadcasted_iota(jnp.int32, (D, H), 0) // DH
    col = jax.lax.broadcasted_iota(jnp.int32, (D, H), 1)
    sel = (row == col).astype(f32)
    logits = dot(qd * k, sel) * (1.0 / np.sqrt(DH))
    ex = jnp.exp(logits)
    return ex, vsc


def _edge_body_x2h(ea_ref, gd_ref, gs_ref, dmod_ref, *refs):
    wk = refs[0:8]
    wv = refs[8:16]
    eww, ewb = refs[16], refs[17]
    exv_ref, den_ref, relp_ref = refs[18], refs[19], refs[20]
    f32 = jnp.float32
    gd = gd_ref[...]
    gs = gs_ref[...]
    rel = gd[:, 2 * D:2 * D + 3] - gs[:, D:D + 3]
    dist = jnp.sqrt(jnp.sum(rel * rel, axis=-1, keepdims=True) + 1e-12)
    relp_ref[...] = jnp.concatenate(
        [rel, dist, jnp.zeros((rel.shape[0], 4), f32)], axis=-1)
    ex, vsc = _edge_common(ea_ref[...], gd[:, 0:D], gd[:, D:2 * D],
                           gs[:, 0:D], rel, dist, wk, wv, eww, ewb)

    row = jax.lax.broadcasted_iota(jnp.int32, (H, D), 0)
    col = jax.lax.broadcasted_iota(jnp.int32, (H, D), 1) // DH
    selt = (row == col).astype(f32)
    exv_ref[...] = jnp.dot(ex, selt, preferred_element_type=f32) * vsc

    dmod = dmod_ref[...]
    zer = jnp.zeros_like(ex)
    den = jnp.zeros((ex.shape[0], D), f32)
    for kk in range(8):
        row_k = jnp.concatenate([zer] * kk + [ex] + [zer] * (7 - kk), axis=-1)
        den = den + jnp.where(dmod == float(kk), row_k, 0.0)
    den_ref[...] = den


def _edge_body_h2x(ea_ref, gd_ref, gs_ref, relp_ref, *refs):
    wk = refs[0:8]
    wv = refs[8:16]
    eww, ewb, rows_ref = refs[16], refs[17], refs[18]
    gd = gd_ref[...]
    relp = relp_ref[...]
    rel = relp[:, 0:3]
    dist = relp[:, 3:4]
    ex, vsc = _edge_common(ea_ref[...], gd[:, 0:D], gd[:, D:2 * D],
                           gs_ref[...], rel, dist, wk, wv, eww, ewb)
    t = ex * vsc
    pieces = [t[:, hh:hh + 1] * rel for hh in range(H)]
    pad = jnp.zeros((ex.shape[0], D - 3 * H - H), jnp.float32)
    rows_ref[...] = jnp.concatenate(pieces + [ex, pad], axis=-1)


def _wsplit(p, dst_is_first):
    bf = jnp.bfloat16
    w1 = p["W1"]
    wa = w1[EF + RF:EF + RF + D]
    wb = w1[EF + RF + D:]
    wd, ws = (wa, wb) if dst_is_first else (wb, wa)
    return (w1[0:EF + RF].astype(bf), wd.astype(bf), ws.astype(bf),
            p["b1"].reshape(1, D), p["g"].reshape(1, D),
            p["bt"].reshape(1, D), p["W2"].astype(bf), p["b2"].reshape(1, -1))


def _edge_pass_x2h(ea, gd, gs, dmod, pk, pv, ew_w, ew_b):
    f32 = jnp.float32
    grid = E // TE

    def im_e(i):
        return (i, 0)

    def im_w(i):
        return (0, 0)

    weights = (*_wsplit(pk, True), *_wsplit(pv, True),
               ew_w.reshape(1, RF), ew_b.reshape(1, 1))
    espec = [pl.BlockSpec((TE, a.shape[1]), im_e) for a in (ea, gd, gs, dmod)]
    wspec = [pl.BlockSpec(w.shape, im_w) for w in weights]
    return pl.pallas_call(
        _edge_body_x2h,
        grid=(grid,),
        in_specs=espec + wspec,
        out_specs=[pl.BlockSpec((TE, D), im_e), pl.BlockSpec((TE, D), im_e),
                   pl.BlockSpec((TE, 8), im_e)],
        out_shape=[jax.ShapeDtypeStruct((E, D), f32),
                   jax.ShapeDtypeStruct((E, D), f32),
                   jax.ShapeDtypeStruct((E, 8), f32)],
    )(ea, gd, gs, dmod, *weights)


def _edge_pass_h2x(ea, gd, gs, relp, pk, pv, ew_w, ew_b):
    f32 = jnp.float32
    grid = E // TE

    def im_e(i):
        return (i, 0)

    def im_w(i):
        return (0, 0)

    weights = (*_wsplit(pk, False), *_wsplit(pv, False),
               ew_w.reshape(1, RF), ew_b.reshape(1, 1))
    espec = [pl.BlockSpec((TE, a.shape[1]), im_e) for a in (ea, gd, gs, relp)]
    wspec = [pl.BlockSpec(w.shape, im_w) for w in weights]
    return pl.pallas_call(
        _edge_body_h2x,
        grid=(grid,),
        in_specs=espec + wspec,
        out_specs=pl.BlockSpec((TE, D), im_e),
        out_shape=jax.ShapeDtypeStruct((E, D), f32),
    )(ea, gd, gs, relp, *weights)


def _mlp(p, x):
    hh = x @ p["W1"] + p["b1"]
    mu = jnp.mean(hh, axis=-1, keepdims=True)
    var = jnp.mean((hh - mu) ** 2, axis=-1, keepdims=True)
    hh = (hh - mu) / jnp.sqrt(var + 1e-5) * p["g"] + p["bt"]
    hh = jax.nn.relu(hh)
    return hh @ p["W2"] + p["b2"]


def kernel(h, x, edge_attr, mask_ligand, params, edge_index):
    f32 = jnp.float32
    dst = edge_index[1]
    src = edge_index[0]
    dst_r = dst.reshape(NW, NCH, GC)
    src_r = src.reshape(NW, NCH, GC)
    dst_g = dst.reshape(NW, NCHG, GCG)
    src_g = src.reshape(NW, NCHG, GCG)
    dden_r = (N + dst // 8).astype(jnp.int32).reshape(NW, NCH, GC)
    dmod = (dst % 8).astype(f32).reshape(E, 1)
    xpad = jnp.pad(x, ((0, 0), (0, XP - 3)))
    zeros1 = jnp.zeros((NACC1 // 16, D), f32)
    zeros2 = jnp.zeros((NACC2 // 16, D), f32)

    p = params["x2h"]
    q1 = _mlp(p["hq"], h)
    td = jnp.concatenate([h, q1, xpad], axis=1)
    ts = jnp.concatenate([h, xpad], axis=1)
    gd, gs = _sc_gather(td, ts, dst_g, src_g)
    exv, denrow, relp = _edge_pass_x2h(edge_attr, gd, gs, dmod,
                                       p["hk"], p["hv"], p["ew_W"], p["ew_b"])
    part = _scatter1(exv, dst_r, denrow, dden_r, zeros1)
    tot = part[0] + part[1]
    num = tot[:N]
    den = tot[N:N + N // 8].reshape(N, H)
    out = (num.reshape(N, H, DH) / (den[:, :, None] + 1e-16)).reshape(N, D)
    out = _mlp(p["node"], jnp.concatenate([out, h], axis=-1))
    h_out = out + h

    p2 = params["h2x"]
    q2 = _mlp(p2["xq"], h_out)
    td2 = jnp.concatenate([h_out, q2], axis=1)
    ts2 = h_out
    gd2, gs2 = _sc_gather(td2, ts2, dst_g, src_g)
    rows2 = _edge_pass_h2x(edge_attr, gd2, gs2, relp,
                           p2["xk"], p2["xv"], p2["ew_W"], p2["ew_b"])
    part2 = _scatter2(rows2, dst_r, zeros2)
    tot2 = part2[0] + part2[1]
    num2 = tot2[:N, :3 * H].reshape(N, H, 3)
    den2 = tot2[:N, 3 * H:4 * H]
    delta = jnp.mean(num2 / (den2[:, :, None] + 1e-16), axis=1)
    x_out = x + delta * mask_ligand[:, None]
    return h_out, x_out
